# baseline (speedup 1.0000x reference)
import functools

import jax
import jax.numpy as jnp
from jax import lax
from jax.experimental import pallas as pl
from jax.experimental.pallas import tpu as pltpu
from jax.experimental.pallas import tpu_sc as plsc

_N = 10000
_E = 160000
_D = 256
_C = 16
_NP = 10112
_KA = 56
_KB = 128
_EPT16 = 10304
_NCH16 = _EPT16 // _KA
_EPT32 = 5248
_NCH32 = _EPT32 // _KB
_RPT = _NP // 16
_R = 1000
_EPS = 1e-5


def _mesh():
    return plsc.VectorSubcoreMesh(
        core_axis_name="c", subcore_axis_name="s", num_cores=2, num_subcores=16
    )


@functools.lru_cache(maxsize=None)
def _make_agg128():
    @functools.partial(
        pl.kernel,
        out_type=jax.ShapeDtypeStruct((2 * _NP, 128), jnp.float32),
        mesh=_mesh(),
        scratch_types=[
            pltpu.VMEM((_NCH16, _KA), jnp.int32),
            pltpu.VMEM((_NCH16, _KA), jnp.int32),
            pltpu.VMEM((_KA, 128), jnp.float32),
            pltpu.VMEM((_KA, 128), jnp.float32),
            pltpu.VMEM((_KA, 128), jnp.float32),
            pltpu.VMEM((_KA, 128), jnp.float32),
            pltpu.VMEM_SHARED((_NP, 128), jnp.float32),
            pltpu.SemaphoreType.DMA,
            pltpu.SemaphoreType.DMA,
            pltpu.SemaphoreType.DMA,
            pltpu.SemaphoreType.DMA,
        ],
        compiler_params=pltpu.CompilerParams(use_tc_tiling_on_sc=False),
    )
    def agg(hcat_hbm, src3_hbm, dst3_hbm, zrows_hbm, out_hbm,
            sidx2, didx2, buf0, buf1, buf2, buf3, acc, sem0, sem1, sem2, sem3):
        cid = lax.axis_index("c")
        sid = lax.axis_index("s")
        wid = cid * 16 + sid
        r0 = sid * _RPT
        pltpu.sync_copy(zrows_hbm, acc.at[pl.ds(r0, _RPT)])
        pltpu.sync_copy(src3_hbm.at[wid], sidx2)
        pltpu.sync_copy(dst3_hbm.at[sid], didx2)
        plsc.subcore_barrier()

        bufs = (buf0, buf1, buf2, buf3)
        sems = (sem0, sem1, sem2, sem3)
        for j in range(4):
            pltpu.make_async_copy(hcat_hbm.at[sidx2.at[j]], bufs[j], sems[j]).start()

        def body(i, carry):
            c0 = 4 * i
            for j in range(4):
                pltpu.make_async_copy(hcat_hbm.at[sidx2.at[c0 + j]], bufs[j], sems[j]).wait()
                pltpu.sync_copy(bufs[j], acc.at[didx2.at[c0 + j]], add=True)
                pltpu.make_async_copy(hcat_hbm.at[sidx2.at[c0 + j + 4]], bufs[j], sems[j]).start()
            return carry

        lax.fori_loop(0, _NCH16 // 4 - 1, body, 0)
        for j in range(4):
            c = _NCH16 - 4 + j
            pltpu.make_async_copy(hcat_hbm.at[sidx2.at[c]], bufs[j], sems[j]).wait()
            pltpu.sync_copy(bufs[j], acc.at[didx2.at[c]], add=True)
        plsc.subcore_barrier()
        pltpu.sync_copy(acc.at[pl.ds(r0, _RPT)], out_hbm.at[pl.ds(cid * _NP + r0, _RPT)])

    return agg


@functools.lru_cache(maxsize=None)
def _make_agg16():
    @functools.partial(
        pl.kernel,
        out_type=jax.ShapeDtypeStruct((2 * _NP, 16), jnp.float32),
        mesh=_mesh(),
        scratch_types=[
            pltpu.VMEM((_NCH32, _KB), jnp.int32),
            pltpu.VMEM((_NCH32, _KB), jnp.int32),
            pltpu.VMEM((_KB, 16), jnp.float32),
            pltpu.VMEM((_KB, 16), jnp.float32),
            pltpu.VMEM_SHARED((_NP, 16), jnp.float32),
            pltpu.SemaphoreType.DMA,
            pltpu.SemaphoreType.DMA,
        ],
        compiler_params=pltpu.CompilerParams(use_tc_tiling_on_sc=False),
    )
    def agg(q_hbm, src3_hbm, dst3_hbm, z16_hbm, out_hbm,
            sidx2, didx2, buf0, buf1, acc, sem0, sem1):
        cid = lax.axis_index("c")
        sid = lax.axis_index("s")
        wid = cid * 16 + sid
        r0 = sid * _RPT
        pltpu.sync_copy(z16_hbm, acc.at[pl.ds(r0, _RPT)])
        pltpu.sync_copy(src3_hbm.at[wid], sidx2)
        pltpu.sync_copy(dst3_hbm.at[wid], didx2)
        plsc.subcore_barrier()

        pltpu.make_async_copy(q_hbm.at[sidx2.at[0]], buf0, sem0).start()

        def body(i, carry):
            c0 = 2 * i
            pltpu.make_async_copy(q_hbm.at[sidx2.at[c0 + 1]], buf1, sem1).start()
            pltpu.make_async_copy(q_hbm.at[sidx2.at[c0]], buf0, sem0).wait()
            pltpu.sync_copy(buf0, acc.at[didx2.at[c0]], add=True)
            pltpu.make_async_copy(q_hbm.at[sidx2.at[c0 + 2]], buf0, sem0).start()
            pltpu.make_async_copy(q_hbm.at[sidx2.at[c0 + 1]], buf1, sem1).wait()
            pltpu.sync_copy(buf1, acc.at[didx2.at[c0 + 1]], add=True)
            return carry

        lax.fori_loop(0, (_NCH32 - 1) // 2, body, 0)
        pltpu.make_async_copy(q_hbm.at[sidx2.at[_NCH32 - 1]], buf0, sem0).wait()
        pltpu.sync_copy(buf0, acc.at[didx2.at[_NCH32 - 1]], add=True)
        plsc.subcore_barrier()
        pltpu.sync_copy(acc.at[pl.ds(r0, _RPT)], out_hbm.at[pl.ds(cid * _NP + r0, _RPT)])

    return agg


@functools.lru_cache(maxsize=None)
def _make_counts():
    @functools.partial(
        pl.kernel,
        out_type=jax.ShapeDtypeStruct((2 * _NP, 16), jnp.float32),
        mesh=_mesh(),
        scratch_types=[
            pltpu.VMEM((_NCH32, _KB), jnp.int32),
            pltpu.VMEM((_KB, 16), jnp.float32),
            pltpu.VMEM_SHARED((_NP, 16), jnp.float32),
        ],
        compiler_params=pltpu.CompilerParams(use_tc_tiling_on_sc=False),
    )
    def cnt(dstp_hbm, ones_hbm, z16_hbm, out_hbm, didx2, ones_v, acc):
        cid = lax.axis_index("c")
        sid = lax.axis_index("s")
        wid = cid * 16 + sid
        r0 = sid * _RPT
        pltpu.sync_copy(z16_hbm, acc.at[pl.ds(r0, _RPT)])
        pltpu.sync_copy(ones_hbm, ones_v)
        pltpu.sync_copy(dstp_hbm.at[wid], didx2)
        plsc.subcore_barrier()

        def body(i, carry):
            pltpu.sync_copy(ones_v, acc.at[didx2.at[i]], add=True)
            return carry

        lax.fori_loop(0, _NCH32, body, 0)
        plsc.subcore_barrier()
        pltpu.sync_copy(acc.at[pl.ds(r0, _RPT)], out_hbm.at[pl.ds(cid * _NP + r0, _RPT)])

    return cnt


def _prew_body(h_ref, wr_ref, bl_ref, out_ref):
    hb = jnp.concatenate([h_ref[0], h_ref[1]], axis=1)
    out_ref[...] = (
        jnp.dot(hb, wr_ref[...], preferred_element_type=jnp.float32) + bl_ref[...]
    )


def _layer_prew(h2, WrT, bl):
    grid = _N // _R
    no = WrT.shape[1]
    return pl.pallas_call(
        _prew_body,
        grid=(grid,),
        in_specs=[
            pl.BlockSpec((2, _R, 128), lambda i: (0, i, 0)),
            pl.BlockSpec((_D, no), lambda i: (0, 0)),
            pl.BlockSpec((1, no), lambda i: (0, 0)),
        ],
        out_specs=pl.BlockSpec((_R, no), lambda i: (i, 0)),
        out_shape=jax.ShapeDtypeStruct((_N, no), jnp.float32),
    )(h2, WrT, bl)


def _pre_body(s_ref, c_ref, pw_ref, wl_ref, pre_ref, s1_ref, s2_ref):
    i = pl.program_id(0)
    sb = jnp.concatenate([s_ref[0], s_ref[1]], axis=1)
    cnt = c_ref[0, :, 0:1] + c_ref[1, :, 0:1]
    recip = 1.0 / jnp.maximum(cnt, 1.0)
    mean = sb * recip
    pre = jnp.dot(mean, wl_ref[...], preferred_element_type=jnp.float32) + pw_ref[...]
    pre_ref[...] = pre
    cs = jnp.sum(pre, axis=0, keepdims=True)
    cq = jnp.sum(pre * pre, axis=0, keepdims=True)

    @pl.when(i == 0)
    def _():
        s1_ref[...] = jnp.zeros_like(s1_ref)
        s2_ref[...] = jnp.zeros_like(s2_ref)

    s1_ref[...] += jnp.broadcast_to(cs, (8, _D))
    s2_ref[...] += jnp.broadcast_to(cq, (8, _D))


def _layer_pre(S2, cnt2, preW, WlT):
    grid = _N // _R
    return pl.pallas_call(
        _pre_body,
        grid=(grid,),
        in_specs=[
            pl.BlockSpec((2, _R, 128), lambda i: (0, i, 0)),
            pl.BlockSpec((2, _R, 16), lambda i: (0, i, 0)),
            pl.BlockSpec((_R, _D), lambda i: (i, 0)),
            pl.BlockSpec((_D, _D), lambda i: (0, 0)),
        ],
        out_specs=[
            pl.BlockSpec((_R, _D), lambda i: (i, 0)),
            pl.BlockSpec((8, _D), lambda i: (0, 0)),
            pl.BlockSpec((8, _D), lambda i: (0, 0)),
        ],
        out_shape=[
            jax.ShapeDtypeStruct((_N, _D), jnp.float32),
            jax.ShapeDtypeStruct((8, _D), jnp.float32),
            jax.ShapeDtypeStruct((8, _D), jnp.float32),
        ],
    )(S2, cnt2, preW, WlT)


def _make_apply_body(has_res, has_q):
    def body(*refs):
        refs = list(refs)
        pre_ref, s1_ref, s2_ref, g_ref, b_ref = refs[:5]
        refs = refs[5:]
        res_ref = refs.pop(0) if has_res else None
        wq_ref = refs.pop(0) if has_q else None
        h_ref = refs.pop(0)
        q_ref = refs.pop(0) if has_q else None

        mu = s1_ref[0:1, :] / float(_N)
        var = s2_ref[0:1, :] / float(_N) - mu * mu
        a = g_ref[...] * lax.rsqrt(var + _EPS)
        c = b_ref[...] - a * mu
        hn = jnp.maximum(pre_ref[...] * a + c, 0.0)
        if has_res:
            hn = hn + jnp.concatenate([res_ref[0], res_ref[1]], axis=1)
        h_ref[...] = jnp.stack([hn[:, :128], hn[:, 128:]], axis=0)
        if has_q:
            q_ref[...] = jnp.dot(hn, wq_ref[...], preferred_element_type=jnp.float32)

    return body


def _layer_apply(pre, s1, s2, g, b, res=None, Wl3T=None):
    grid = _N // _R
    has_res = res is not None
    has_q = Wl3T is not None
    in_specs = [
        pl.BlockSpec((_R, _D), lambda i: (i, 0)),
        pl.BlockSpec((8, _D), lambda i: (0, 0)),
        pl.BlockSpec((8, _D), lambda i: (0, 0)),
        pl.BlockSpec((1, _D), lambda i: (0, 0)),
        pl.BlockSpec((1, _D), lambda i: (0, 0)),
    ]
    args = [pre, s1, s2, g, b]
    if has_res:
        in_specs.append(pl.BlockSpec((2, _R, 128), lambda i: (0, i, 0)))
        args.append(res)
    if has_q:
        in_specs.append(pl.BlockSpec((_D, _C), lambda i: (0, 0)))
        args.append(Wl3T)
    out_specs = [pl.BlockSpec((2, _R, 128), lambda i: (0, i, 0))]
    out_shape = [jax.ShapeDtypeStruct((2, _N, 128), jnp.float32)]
    if has_q:
        out_specs.append(pl.BlockSpec((_R, _C), lambda i: (i, 0)))
        out_shape.append(jax.ShapeDtypeStruct((_N, _C), jnp.float32))
    out = pl.pallas_call(
        _make_apply_body(has_res, has_q),
        grid=(grid,),
        in_specs=in_specs,
        out_specs=out_specs,
        out_shape=out_shape,
    )(*args)
    return out if has_q else out[0]


def _final_body(t_ref, c_ref, fw_ref, out_ref):
    cnt = c_ref[0, :, 0:1] + c_ref[1, :, 0:1]
    recip = 1.0 / jnp.maximum(cnt, 1.0)
    tsum = t_ref[0] + t_ref[1]
    out_ref[...] = tsum * recip + fw_ref[...]


def _layer_final(T2, cnt2, fW):
    grid = _N // _R
    return pl.pallas_call(
        _final_body,
        grid=(grid,),
        in_specs=[
            pl.BlockSpec((2, _R, 16), lambda i: (0, i, 0)),
            pl.BlockSpec((2, _R, 16), lambda i: (0, i, 0)),
            pl.BlockSpec((_R, _C), lambda i: (i, 0)),
        ],
        out_specs=pl.BlockSpec((_R, _C), lambda i: (i, 0)),
        out_shape=jax.ShapeDtypeStruct((_N, _C), jnp.float32),
    )(T2, cnt2, fW)


def kernel(x, edge_index, Wl0, bl0, Wr0, Wl1, bl1, Wr1, Wl2, bl2, Wr2, Wl3, bl3, Wr3, g0, b0, g1, b1, g2, b2):
    f32 = jnp.float32
    src = edge_index[0]
    dst = edge_index[1]
    pad16 = 16 * _EPT16 - _E
    pad32 = 32 * _EPT32 - _E
    srcp = jnp.concatenate([src, jnp.zeros((pad16,), jnp.int32)])
    dstp = jnp.concatenate([dst, jnp.full((pad16,), _N, jnp.int32)])
    s16 = srcp.reshape(16, _NCH16, _KA)
    src3 = jnp.concatenate([s16, s16 + _N], axis=0)
    dst3 = dstp.reshape(16, _NCH16, _KA)
    src32 = jnp.concatenate([src, jnp.zeros((pad32,), jnp.int32)]).reshape(32, _NCH32, _KB)
    dst32 = jnp.concatenate([dst, jnp.full((pad32,), _N, jnp.int32)]).reshape(32, _NCH32, _KB)
    zrows = jnp.zeros((_RPT, 128), f32)
    z16 = jnp.zeros((_RPT, 16), f32)
    ones16 = jnp.ones((_KB, 16), f32)

    cnt2 = _make_counts()(dst32, ones16, z16).reshape(2, _NP, 16)

    WlTs = [Wl0.T, Wl1.T, Wl2.T]
    WrTs = [Wr0.T, Wr1.T, Wr2.T]
    bls = [bl0.reshape(1, _D), bl1.reshape(1, _D), bl2.reshape(1, _D)]
    gs = [g0.reshape(1, _D), g1.reshape(1, _D), g2.reshape(1, _D)]
    bs = [b0.reshape(1, _D), b1.reshape(1, _D), b2.reshape(1, _D)]

    h = jnp.stack([x[:, :128], x[:, 128:]], axis=0)
    q = None
    for l in range(3):
        S2 = _make_agg128()(h.reshape(2 * _N, 128), src3, dst3, zrows).reshape(2, _NP, 128)
        preW = _layer_prew(h, WrTs[l], bls[l])
        pre, s1, s2 = _layer_pre(S2, cnt2, preW, WlTs[l])
        if l < 2:
            h = _layer_apply(pre, s1, s2, gs[l], bs[l], res=(h if l > 0 else None))
        else:
            h, q = _layer_apply(pre, s1, s2, gs[l], bs[l], res=h, Wl3T=Wl3.T)

    T2 = _make_agg16()(q, src32, dst32, z16).reshape(2, _NP, 16)
    fW = _layer_prew(h, Wr3.T, bl3.reshape(1, _C))
    return _layer_final(T2, cnt2, fW)


# speedup vs baseline: 1.2508x; 1.2508x over previous
import functools

import jax
import jax.numpy as jnp
from jax import lax
from jax.experimental import pallas as pl
from jax.experimental.pallas import tpu as pltpu
from jax.experimental.pallas import tpu_sc as plsc

_N = 10000
_E = 160000
_D = 256
_C = 16
_NP = 10112
_KA = 128
_KB = 128
_EPT16 = 10112
_NCH16 = _EPT16 // _KA
_EPT32 = 5248
_NCH32 = _EPT32 // _KB
_RPT = _NP // 16
_R = 1000
_EPS = 1e-5

import numpy as _np
_g = _np.arange(_D) // 32 * 32
_o = _np.arange(_D) % 32
_PERM = _np.where(_o < 16, _g + 2 * _o, _g + 2 * (_o - 16) + 1)


def _mesh():
    return plsc.VectorSubcoreMesh(
        core_axis_name="c", subcore_axis_name="s", num_cores=2, num_subcores=16
    )


@functools.lru_cache(maxsize=None)
def _make_agg128():
    @functools.partial(
        pl.kernel,
        out_type=jax.ShapeDtypeStruct((2 * _NP, 128), jnp.float32),
        mesh=_mesh(),
        scratch_types=[
            pltpu.VMEM((_NCH16, _KA), jnp.int16),
            pltpu.VMEM((_NCH16, _KA), jnp.int16),
            pltpu.VMEM((_KA,), jnp.int32),
            pltpu.VMEM((_KA,), jnp.int32),
            pltpu.VMEM((_KA,), jnp.int32),
            pltpu.VMEM((_KA, 128), jnp.bfloat16),
            pltpu.VMEM((_KA, 128), jnp.bfloat16),
            pltpu.VMEM((_KA, 128), jnp.float32),
            pltpu.VMEM_SHARED((_NP, 128), jnp.float32),
            pltpu.SemaphoreType.DMA,
            pltpu.SemaphoreType.DMA,
        ],
        compiler_params=pltpu.CompilerParams(
            use_tc_tiling_on_sc=False, needs_layout_passes=False
        ),
    )
    def agg(hcat_hbm, src3_hbm, dst3_hbm, zrows_hbm, out_hbm,
            sidx16, didx16, sstg0, sstg1, dstg, b0, b1, fs, acc, sem0, sem1):
        cid = lax.axis_index("c")
        sid = lax.axis_index("s")
        wid = cid * 16 + sid
        r0 = sid * _RPT
        pltpu.sync_copy(zrows_hbm, acc.at[pl.ds(r0, _RPT)])
        pltpu.sync_copy(src3_hbm.at[wid], sidx16)
        pltpu.sync_copy(dst3_hbm.at[sid], didx16)
        plsc.subcore_barrier()

        def cvt_idx(tbl, c, out_ref):
            for g in range(_KA // 32):
                v = tbl[c, pl.ds(32 * g, 32)]
                a, b = plsc.unpack(v, format=plsc.PackFormat.INTERLEAVED)
                out_ref[pl.ds(32 * g, 16)] = a
                out_ref[pl.ds(32 * g + 16, 16)] = b

        def cvt_rows(bsrc):
            def rbody(r, carry):
                for g in range(4):
                    v = bsrc[r, pl.ds(32 * g, 32)]
                    a, b = plsc.unpack(v, format=plsc.PackFormat.INTERLEAVED)
                    fs[r, pl.ds(32 * g, 16)] = a
                    fs[r, pl.ds(32 * g + 16, 16)] = b
                return carry
            lax.fori_loop(0, _KA, rbody, 0)

        def gather(stg, buf, sem):
            pltpu.make_async_copy(hcat_hbm.at[stg], buf, sem).start()

        def handle(buf, sem, c, stg):
            pltpu.make_async_copy(hcat_hbm.at[stg], buf, sem).wait()
            cvt_rows(buf)
            cvt_idx(didx16, c, dstg)
            pltpu.sync_copy(fs, acc.at[dstg], add=True)

        cvt_idx(sidx16, 0, sstg0)
        gather(sstg0, b0, sem0)
        cvt_idx(sidx16, 1, sstg1)
        gather(sstg1, b1, sem1)

        def body(i, carry):
            c0 = 2 * i
            handle(b0, sem0, c0, sstg0)
            cvt_idx(sidx16, c0 + 2, sstg0)
            gather(sstg0, b0, sem0)
            handle(b1, sem1, c0 + 1, sstg1)
            cvt_idx(sidx16, c0 + 3, sstg1)
            gather(sstg1, b1, sem1)
            return carry

        lax.fori_loop(0, (_NCH16 - 3) // 2, body, 0)
        handle(b0, sem0, _NCH16 - 3, sstg0)
        cvt_idx(sidx16, _NCH16 - 1, sstg0)
        gather(sstg0, b0, sem0)
        handle(b1, sem1, _NCH16 - 2, sstg1)
        handle(b0, sem0, _NCH16 - 1, sstg0)
        plsc.subcore_barrier()
        pltpu.sync_copy(acc.at[pl.ds(r0, _RPT)], out_hbm.at[pl.ds(cid * _NP + r0, _RPT)])

    return agg


@functools.lru_cache(maxsize=None)
def _make_agg16():
    @functools.partial(
        pl.kernel,
        out_type=jax.ShapeDtypeStruct((2 * _NP, 16), jnp.float32),
        mesh=_mesh(),
        scratch_types=[
            pltpu.VMEM((_NCH32, _KB), jnp.int32),
            pltpu.VMEM((_NCH32, _KB), jnp.int32),
            pltpu.VMEM((_KB, 16), jnp.float32),
            pltpu.VMEM((_KB, 16), jnp.float32),
            pltpu.VMEM_SHARED((_NP, 16), jnp.float32),
            pltpu.SemaphoreType.DMA,
            pltpu.SemaphoreType.DMA,
        ],
        compiler_params=pltpu.CompilerParams(use_tc_tiling_on_sc=False),
    )
    def agg(q_hbm, src3_hbm, dst3_hbm, z16_hbm, out_hbm,
            sidx2, didx2, buf0, buf1, acc, sem0, sem1):
        cid = lax.axis_index("c")
        sid = lax.axis_index("s")
        wid = cid * 16 + sid
        r0 = sid * _RPT
        pltpu.sync_copy(z16_hbm, acc.at[pl.ds(r0, _RPT)])
        pltpu.sync_copy(src3_hbm.at[wid], sidx2)
        pltpu.sync_copy(dst3_hbm.at[wid], didx2)
        plsc.subcore_barrier()

        pltpu.make_async_copy(q_hbm.at[sidx2.at[0]], buf0, sem0).start()

        def body(i, carry):
            c0 = 2 * i
            pltpu.make_async_copy(q_hbm.at[sidx2.at[c0 + 1]], buf1, sem1).start()
            pltpu.make_async_copy(q_hbm.at[sidx2.at[c0]], buf0, sem0).wait()
            pltpu.sync_copy(buf0, acc.at[didx2.at[c0]], add=True)
            pltpu.make_async_copy(q_hbm.at[sidx2.at[c0 + 2]], buf0, sem0).start()
            pltpu.make_async_copy(q_hbm.at[sidx2.at[c0 + 1]], buf1, sem1).wait()
            pltpu.sync_copy(buf1, acc.at[didx2.at[c0 + 1]], add=True)
            return carry

        lax.fori_loop(0, (_NCH32 - 1) // 2, body, 0)
        pltpu.make_async_copy(q_hbm.at[sidx2.at[_NCH32 - 1]], buf0, sem0).wait()
        pltpu.sync_copy(buf0, acc.at[didx2.at[_NCH32 - 1]], add=True)
        plsc.subcore_barrier()
        pltpu.sync_copy(acc.at[pl.ds(r0, _RPT)], out_hbm.at[pl.ds(cid * _NP + r0, _RPT)])

    return agg


@functools.lru_cache(maxsize=None)
def _make_counts():
    @functools.partial(
        pl.kernel,
        out_type=jax.ShapeDtypeStruct((2 * _NP, 16), jnp.float32),
        mesh=_mesh(),
        scratch_types=[
            pltpu.VMEM((_NCH32, _KB), jnp.int32),
            pltpu.VMEM((_KB, 16), jnp.float32),
            pltpu.VMEM_SHARED((_NP, 16), jnp.float32),
        ],
        compiler_params=pltpu.CompilerParams(use_tc_tiling_on_sc=False),
    )
    def cnt(dstp_hbm, ones_hbm, z16_hbm, out_hbm, didx2, ones_v, acc):
        cid = lax.axis_index("c")
        sid = lax.axis_index("s")
        wid = cid * 16 + sid
        r0 = sid * _RPT
        pltpu.sync_copy(z16_hbm, acc.at[pl.ds(r0, _RPT)])
        pltpu.sync_copy(ones_hbm, ones_v)
        pltpu.sync_copy(dstp_hbm.at[wid], didx2)
        plsc.subcore_barrier()

        def body(i, carry):
            pltpu.sync_copy(ones_v, acc.at[didx2.at[i]], add=True)
            return carry

        lax.fori_loop(0, _NCH32, body, 0)
        plsc.subcore_barrier()
        pltpu.sync_copy(acc.at[pl.ds(r0, _RPT)], out_hbm.at[pl.ds(cid * _NP + r0, _RPT)])

    return cnt


def _prew_body(h_ref, wr_ref, bl_ref, out_ref):
    hb = jnp.concatenate([h_ref[0], h_ref[1]], axis=1)
    out_ref[...] = (
        jnp.dot(hb, wr_ref[...], preferred_element_type=jnp.float32) + bl_ref[...]
    )


def _layer_prew(h2, WrT, bl):
    grid = _N // _R
    no = WrT.shape[1]
    return pl.pallas_call(
        _prew_body,
        grid=(grid,),
        in_specs=[
            pl.BlockSpec((2, _R, 128), lambda i: (0, i, 0)),
            pl.BlockSpec((_D, no), lambda i: (0, 0)),
            pl.BlockSpec((1, no), lambda i: (0, 0)),
        ],
        out_specs=pl.BlockSpec((_R, no), lambda i: (i, 0)),
        out_shape=jax.ShapeDtypeStruct((_N, no), jnp.float32),
    )(h2, WrT, bl)


def _pre_body(s_ref, c_ref, pw_ref, wl_ref, pre_ref, s1_ref, s2_ref):
    i = pl.program_id(0)
    sb = jnp.concatenate([s_ref[0], s_ref[1]], axis=1)
    cnt = c_ref[0, :, 0:1] + c_ref[1, :, 0:1]
    recip = 1.0 / jnp.maximum(cnt, 1.0)
    mean = sb * recip
    pre = jnp.dot(mean, wl_ref[...], preferred_element_type=jnp.float32) + pw_ref[...]
    pre_ref[...] = pre
    cs = jnp.sum(pre, axis=0, keepdims=True)
    cq = jnp.sum(pre * pre, axis=0, keepdims=True)

    @pl.when(i == 0)
    def _():
        s1_ref[...] = jnp.zeros_like(s1_ref)
        s2_ref[...] = jnp.zeros_like(s2_ref)

    s1_ref[...] += jnp.broadcast_to(cs, (8, _D))
    s2_ref[...] += jnp.broadcast_to(cq, (8, _D))


def _layer_pre(S2, cnt2, preW, WlT):
    grid = _N // _R
    return pl.pallas_call(
        _pre_body,
        grid=(grid,),
        in_specs=[
            pl.BlockSpec((2, _R, 128), lambda i: (0, i, 0)),
            pl.BlockSpec((2, _R, 16), lambda i: (0, i, 0)),
            pl.BlockSpec((_R, _D), lambda i: (i, 0)),
            pl.BlockSpec((_D, _D), lambda i: (0, 0)),
        ],
        out_specs=[
            pl.BlockSpec((_R, _D), lambda i: (i, 0)),
            pl.BlockSpec((8, _D), lambda i: (0, 0)),
            pl.BlockSpec((8, _D), lambda i: (0, 0)),
        ],
        out_shape=[
            jax.ShapeDtypeStruct((_N, _D), jnp.float32),
            jax.ShapeDtypeStruct((8, _D), jnp.float32),
            jax.ShapeDtypeStruct((8, _D), jnp.float32),
        ],
    )(S2, cnt2, preW, WlT)


def _make_apply_body(has_res, has_q):
    def body(*refs):
        refs = list(refs)
        pre_ref, s1_ref, s2_ref, g_ref, b_ref = refs[:5]
        refs = refs[5:]
        res_ref = refs.pop(0) if has_res else None
        wq_ref = refs.pop(0) if has_q else None
        h_ref = refs.pop(0)
        h16_ref = refs.pop(0)
        q_ref = refs.pop(0) if has_q else None

        mu = s1_ref[0:1, :] / float(_N)
        var = s2_ref[0:1, :] / float(_N) - mu * mu
        a = g_ref[...] * lax.rsqrt(var + _EPS)
        c = b_ref[...] - a * mu
        hn = jnp.maximum(pre_ref[...] * a + c, 0.0)
        if has_res:
            hn = hn + jnp.concatenate([res_ref[0], res_ref[1]], axis=1)
        stacked = jnp.stack([hn[:, :128], hn[:, 128:]], axis=0)
        h_ref[...] = stacked
        h16_ref[...] = stacked.astype(jnp.bfloat16)
        if has_q:
            q_ref[...] = jnp.dot(hn, wq_ref[...], preferred_element_type=jnp.float32)

    return body


def _layer_apply(pre, s1, s2, g, b, res=None, Wl3T=None):
    grid = _N // _R
    has_res = res is not None
    has_q = Wl3T is not None
    in_specs = [
        pl.BlockSpec((_R, _D), lambda i: (i, 0)),
        pl.BlockSpec((8, _D), lambda i: (0, 0)),
        pl.BlockSpec((8, _D), lambda i: (0, 0)),
        pl.BlockSpec((1, _D), lambda i: (0, 0)),
        pl.BlockSpec((1, _D), lambda i: (0, 0)),
    ]
    args = [pre, s1, s2, g, b]
    if has_res:
        in_specs.append(pl.BlockSpec((2, _R, 128), lambda i: (0, i, 0)))
        args.append(res)
    if has_q:
        in_specs.append(pl.BlockSpec((_D, _C), lambda i: (0, 0)))
        args.append(Wl3T)
    out_specs = [
        pl.BlockSpec((2, _R, 128), lambda i: (0, i, 0)),
        pl.BlockSpec((2, _R, 128), lambda i: (0, i, 0)),
    ]
    out_shape = [
        jax.ShapeDtypeStruct((2, _N, 128), jnp.float32),
        jax.ShapeDtypeStruct((2, _N, 128), jnp.bfloat16),
    ]
    if has_q:
        out_specs.append(pl.BlockSpec((_R, _C), lambda i: (i, 0)))
        out_shape.append(jax.ShapeDtypeStruct((_N, _C), jnp.float32))
    return pl.pallas_call(
        _make_apply_body(has_res, has_q),
        grid=(grid,),
        in_specs=in_specs,
        out_specs=out_specs,
        out_shape=out_shape,
    )(*args)


def _final_body(t_ref, c_ref, fw_ref, out_ref):
    cnt = c_ref[0, :, 0:1] + c_ref[1, :, 0:1]
    recip = 1.0 / jnp.maximum(cnt, 1.0)
    tsum = t_ref[0] + t_ref[1]
    out_ref[...] = tsum * recip + fw_ref[...]


def _layer_final(T2, cnt2, fW):
    grid = _N // _R
    return pl.pallas_call(
        _final_body,
        grid=(grid,),
        in_specs=[
            pl.BlockSpec((2, _R, 16), lambda i: (0, i, 0)),
            pl.BlockSpec((2, _R, 16), lambda i: (0, i, 0)),
            pl.BlockSpec((_R, _C), lambda i: (i, 0)),
        ],
        out_specs=pl.BlockSpec((_R, _C), lambda i: (i, 0)),
        out_shape=jax.ShapeDtypeStruct((_N, _C), jnp.float32),
    )(T2, cnt2, fW)


def kernel(x, edge_index, Wl0, bl0, Wr0, Wl1, bl1, Wr1, Wl2, bl2, Wr2, Wl3, bl3, Wr3, g0, b0, g1, b1, g2, b2):
    f32 = jnp.float32
    src = edge_index[0]
    dst = edge_index[1]
    pad16 = 16 * _EPT16 - _E
    pad32 = 32 * _EPT32 - _E
    srcp = jnp.concatenate([src, jnp.zeros((pad16,), jnp.int32)])
    dstp = jnp.concatenate([dst, jnp.full((pad16,), _N, jnp.int32)])
    s16 = srcp.reshape(16, _NCH16, _KA)
    src3 = jnp.concatenate([s16, s16 + _N], axis=0).astype(jnp.int16)
    dst3 = dstp.reshape(16, _NCH16, _KA).astype(jnp.int16)
    src32 = jnp.concatenate([src, jnp.zeros((pad32,), jnp.int32)]).reshape(32, _NCH32, _KB)
    dst32 = jnp.concatenate([dst, jnp.full((pad32,), _N, jnp.int32)]).reshape(32, _NCH32, _KB)
    zrows = jnp.zeros((_RPT, 128), f32)
    z16 = jnp.zeros((_RPT, 16), f32)
    ones16 = jnp.ones((_KB, 16), f32)

    cnt2 = _make_counts()(dst32, ones16, z16).reshape(2, _NP, 16)

    perm = jnp.asarray(_PERM)
    WlTs = [Wl0.T[perm], Wl1.T[perm], Wl2.T[perm]]
    WrTs = [Wr0.T, Wr1.T, Wr2.T]
    bls = [bl0.reshape(1, _D), bl1.reshape(1, _D), bl2.reshape(1, _D)]
    gs = [g0.reshape(1, _D), g1.reshape(1, _D), g2.reshape(1, _D)]
    bs = [b0.reshape(1, _D), b1.reshape(1, _D), b2.reshape(1, _D)]

    h = jnp.stack([x[:, :128], x[:, 128:]], axis=0)
    h16 = h.astype(jnp.bfloat16)
    q = None
    for l in range(3):
        S2 = _make_agg128()(h16.reshape(2 * _N, 128), src3, dst3, zrows).reshape(2, _NP, 128)
        preW = _layer_prew(h, WrTs[l], bls[l])
        pre, s1, s2 = _layer_pre(S2, cnt2, preW, WlTs[l])
        if l < 2:
            h, h16 = _layer_apply(pre, s1, s2, gs[l], bs[l], res=(h if l > 0 else None))
        else:
            h, h16, q = _layer_apply(pre, s1, s2, gs[l], bs[l], res=h, Wl3T=Wl3.T)

    T2 = _make_agg16()(q, src32, dst32, z16).reshape(2, _NP, 16)
    fW = _layer_prew(h, Wr3.T, bl3.reshape(1, _C))
    return _layer_final(T2, cnt2, fW)


# speedup vs baseline: 1.8214x; 1.4561x over previous
import functools

import jax
import jax.numpy as jnp
from jax import lax
from jax.experimental import pallas as pl
from jax.experimental.pallas import tpu as pltpu
from jax.experimental.pallas import tpu_sc as plsc

_N = 10000
_E = 160000
_D = 256
_C = 16
_NP = 10112
_KA = 128
_KB = 128
_EPT16 = 10112
_NCH16 = _EPT16 // _KA
_EPT32 = 5248
_NCH32 = _EPT32 // _KB
_RPT = _NP // 16
_R = 1000
_EPS = 1e-5

import numpy as _np
_g = _np.arange(_D) // 32 * 32
_o = _np.arange(_D) % 32
_PERM = _np.where(_o < 16, _g + 2 * _o, _g + 2 * (_o - 16) + 1)


def _mesh():
    return plsc.VectorSubcoreMesh(
        core_axis_name="c", subcore_axis_name="s", num_cores=2, num_subcores=16
    )


@functools.lru_cache(maxsize=None)
def _make_agg128():
    @functools.partial(
        pl.kernel,
        out_type=jax.ShapeDtypeStruct((2 * _NP, 128), jnp.float32),
        mesh=_mesh(),
        scratch_types=[
            pltpu.VMEM((_NCH16, _KA), jnp.int16),
            pltpu.VMEM((_NCH16, _KA), jnp.int16),
            pltpu.VMEM((_KA,), jnp.int32),
            pltpu.VMEM((_KA,), jnp.int32),
            pltpu.VMEM((_KA,), jnp.int32),
            pltpu.VMEM((_KA, 128), jnp.bfloat16),
            pltpu.VMEM((_KA, 128), jnp.bfloat16),
            pltpu.VMEM((_KA, 128), jnp.float32),
            pltpu.VMEM_SHARED((_NP, 128), jnp.float32),
            pltpu.SemaphoreType.DMA,
            pltpu.SemaphoreType.DMA,
        ],
        compiler_params=pltpu.CompilerParams(
            use_tc_tiling_on_sc=False, needs_layout_passes=False
        ),
    )
    def agg(hcat_hbm, src3_hbm, dst3_hbm, zrows_hbm, out_hbm,
            sidx16, didx16, sstg0, sstg1, dstg, b0, b1, fs, acc, sem0, sem1):
        cid = lax.axis_index("c")
        sid = lax.axis_index("s")
        wid = cid * 16 + sid
        r0 = sid * _RPT
        pltpu.sync_copy(zrows_hbm, acc.at[pl.ds(r0, _RPT)])
        pltpu.sync_copy(src3_hbm.at[wid], sidx16)
        pltpu.sync_copy(dst3_hbm.at[sid], didx16)
        plsc.subcore_barrier()

        def cvt_idx(tbl, c, out_ref):
            for g in range(_KA // 32):
                v = tbl[c, pl.ds(32 * g, 32)]
                a, b = plsc.unpack(v, format=plsc.PackFormat.INTERLEAVED)
                out_ref[pl.ds(32 * g, 16)] = a
                out_ref[pl.ds(32 * g + 16, 16)] = b

        def cvt_rows(bsrc):
            @plsc.parallel_loop(0, _KA, unroll=4)
            def rbody(r):
                for g in range(4):
                    v = bsrc[r, pl.ds(32 * g, 32)]
                    a, b = plsc.unpack(v, format=plsc.PackFormat.INTERLEAVED)
                    fs[r, pl.ds(32 * g, 16)] = a
                    fs[r, pl.ds(32 * g + 16, 16)] = b

        def gather(stg, buf, sem):
            pltpu.make_async_copy(hcat_hbm.at[stg], buf, sem).start()

        def handle(buf, sem, c, stg):
            pltpu.make_async_copy(hcat_hbm.at[stg], buf, sem).wait()
            cvt_rows(buf)
            cvt_idx(didx16, c, dstg)
            pltpu.sync_copy(fs, acc.at[dstg], add=True)

        cvt_idx(sidx16, 0, sstg0)
        gather(sstg0, b0, sem0)
        cvt_idx(sidx16, 1, sstg1)
        gather(sstg1, b1, sem1)

        def body(i, carry):
            c0 = 2 * i
            handle(b0, sem0, c0, sstg0)
            cvt_idx(sidx16, c0 + 2, sstg0)
            gather(sstg0, b0, sem0)
            handle(b1, sem1, c0 + 1, sstg1)
            cvt_idx(sidx16, c0 + 3, sstg1)
            gather(sstg1, b1, sem1)
            return carry

        lax.fori_loop(0, (_NCH16 - 3) // 2, body, 0)
        handle(b0, sem0, _NCH16 - 3, sstg0)
        cvt_idx(sidx16, _NCH16 - 1, sstg0)
        gather(sstg0, b0, sem0)
        handle(b1, sem1, _NCH16 - 2, sstg1)
        handle(b0, sem0, _NCH16 - 1, sstg0)
        plsc.subcore_barrier()
        pltpu.sync_copy(acc.at[pl.ds(r0, _RPT)], out_hbm.at[pl.ds(cid * _NP + r0, _RPT)])

    return agg


@functools.lru_cache(maxsize=None)
def _make_agg16():
    @functools.partial(
        pl.kernel,
        out_type=jax.ShapeDtypeStruct((2 * _NP, 16), jnp.float32),
        mesh=_mesh(),
        scratch_types=[
            pltpu.VMEM((_NCH32, _KB), jnp.int32),
            pltpu.VMEM((_NCH32, _KB), jnp.int32),
            pltpu.VMEM((_KB, 16), jnp.float32),
            pltpu.VMEM((_KB, 16), jnp.float32),
            pltpu.VMEM_SHARED((_NP, 16), jnp.float32),
            pltpu.SemaphoreType.DMA,
            pltpu.SemaphoreType.DMA,
        ],
        compiler_params=pltpu.CompilerParams(use_tc_tiling_on_sc=False),
    )
    def agg(q_hbm, src3_hbm, dst3_hbm, z16_hbm, out_hbm,
            sidx2, didx2, buf0, buf1, acc, sem0, sem1):
        cid = lax.axis_index("c")
        sid = lax.axis_index("s")
        wid = cid * 16 + sid
        r0 = sid * _RPT
        pltpu.sync_copy(z16_hbm, acc.at[pl.ds(r0, _RPT)])
        pltpu.sync_copy(src3_hbm.at[wid], sidx2)
        pltpu.sync_copy(dst3_hbm.at[wid], didx2)
        plsc.subcore_barrier()

        pltpu.make_async_copy(q_hbm.at[sidx2.at[0]], buf0, sem0).start()

        def body(i, carry):
            c0 = 2 * i
            pltpu.make_async_copy(q_hbm.at[sidx2.at[c0 + 1]], buf1, sem1).start()
            pltpu.make_async_copy(q_hbm.at[sidx2.at[c0]], buf0, sem0).wait()
            pltpu.sync_copy(buf0, acc.at[didx2.at[c0]], add=True)
            pltpu.make_async_copy(q_hbm.at[sidx2.at[c0 + 2]], buf0, sem0).start()
            pltpu.make_async_copy(q_hbm.at[sidx2.at[c0 + 1]], buf1, sem1).wait()
            pltpu.sync_copy(buf1, acc.at[didx2.at[c0 + 1]], add=True)
            return carry

        lax.fori_loop(0, (_NCH32 - 1) // 2, body, 0)
        pltpu.make_async_copy(q_hbm.at[sidx2.at[_NCH32 - 1]], buf0, sem0).wait()
        pltpu.sync_copy(buf0, acc.at[didx2.at[_NCH32 - 1]], add=True)
        plsc.subcore_barrier()
        pltpu.sync_copy(acc.at[pl.ds(r0, _RPT)], out_hbm.at[pl.ds(cid * _NP + r0, _RPT)])

    return agg


@functools.lru_cache(maxsize=None)
def _make_counts():
    @functools.partial(
        pl.kernel,
        out_type=jax.ShapeDtypeStruct((2 * _NP, 16), jnp.float32),
        mesh=_mesh(),
        scratch_types=[
            pltpu.VMEM((_NCH32, _KB), jnp.int32),
            pltpu.VMEM((_KB, 16), jnp.float32),
            pltpu.VMEM_SHARED((_NP, 16), jnp.float32),
        ],
        compiler_params=pltpu.CompilerParams(use_tc_tiling_on_sc=False),
    )
    def cnt(dstp_hbm, ones_hbm, z16_hbm, out_hbm, didx2, ones_v, acc):
        cid = lax.axis_index("c")
        sid = lax.axis_index("s")
        wid = cid * 16 + sid
        r0 = sid * _RPT
        pltpu.sync_copy(z16_hbm, acc.at[pl.ds(r0, _RPT)])
        pltpu.sync_copy(ones_hbm, ones_v)
        pltpu.sync_copy(dstp_hbm.at[wid], didx2)
        plsc.subcore_barrier()

        def body(i, carry):
            pltpu.sync_copy(ones_v, acc.at[didx2.at[i]], add=True)
            return carry

        lax.fori_loop(0, _NCH32, body, 0)
        plsc.subcore_barrier()
        pltpu.sync_copy(acc.at[pl.ds(r0, _RPT)], out_hbm.at[pl.ds(cid * _NP + r0, _RPT)])

    return cnt


def _prew_body(h_ref, wr_ref, bl_ref, out_ref):
    hb = jnp.concatenate([h_ref[0], h_ref[1]], axis=1)
    out_ref[...] = (
        jnp.dot(hb, wr_ref[...], preferred_element_type=jnp.float32) + bl_ref[...]
    )


def _layer_prew(h2, WrT, bl):
    grid = _N // _R
    no = WrT.shape[1]
    return pl.pallas_call(
        _prew_body,
        grid=(grid,),
        in_specs=[
            pl.BlockSpec((2, _R, 128), lambda i: (0, i, 0)),
            pl.BlockSpec((_D, no), lambda i: (0, 0)),
            pl.BlockSpec((1, no), lambda i: (0, 0)),
        ],
        out_specs=pl.BlockSpec((_R, no), lambda i: (i, 0)),
        out_shape=jax.ShapeDtypeStruct((_N, no), jnp.float32),
    )(h2, WrT, bl)


def _pre_body(s_ref, c_ref, pw_ref, wl_ref, pre_ref, s1_ref, s2_ref):
    i = pl.program_id(0)
    sb = jnp.concatenate([s_ref[0], s_ref[1]], axis=1)
    cnt = c_ref[0, :, 0:1] + c_ref[1, :, 0:1]
    recip = 1.0 / jnp.maximum(cnt, 1.0)
    mean = sb * recip
    pre = jnp.dot(mean, wl_ref[...], preferred_element_type=jnp.float32) + pw_ref[...]
    pre_ref[...] = pre
    cs = jnp.sum(pre, axis=0, keepdims=True)
    cq = jnp.sum(pre * pre, axis=0, keepdims=True)

    @pl.when(i == 0)
    def _():
        s1_ref[...] = jnp.zeros_like(s1_ref)
        s2_ref[...] = jnp.zeros_like(s2_ref)

    s1_ref[...] += jnp.broadcast_to(cs, (8, _D))
    s2_ref[...] += jnp.broadcast_to(cq, (8, _D))


def _layer_pre(S2, cnt2, preW, WlT):
    grid = _N // _R
    return pl.pallas_call(
        _pre_body,
        grid=(grid,),
        in_specs=[
            pl.BlockSpec((2, _R, 128), lambda i: (0, i, 0)),
            pl.BlockSpec((2, _R, 16), lambda i: (0, i, 0)),
            pl.BlockSpec((_R, _D), lambda i: (i, 0)),
            pl.BlockSpec((_D, _D), lambda i: (0, 0)),
        ],
        out_specs=[
            pl.BlockSpec((_R, _D), lambda i: (i, 0)),
            pl.BlockSpec((8, _D), lambda i: (0, 0)),
            pl.BlockSpec((8, _D), lambda i: (0, 0)),
        ],
        out_shape=[
            jax.ShapeDtypeStruct((_N, _D), jnp.float32),
            jax.ShapeDtypeStruct((8, _D), jnp.float32),
            jax.ShapeDtypeStruct((8, _D), jnp.float32),
        ],
    )(S2, cnt2, preW, WlT)


def _make_apply_body(has_res, has_q):
    def body(*refs):
        refs = list(refs)
        pre_ref, s1_ref, s2_ref, g_ref, b_ref = refs[:5]
        refs = refs[5:]
        res_ref = refs.pop(0) if has_res else None
        wq_ref = refs.pop(0) if has_q else None
        h_ref = refs.pop(0)
        h16_ref = refs.pop(0)
        q_ref = refs.pop(0) if has_q else None

        mu = s1_ref[0:1, :] / float(_N)
        var = s2_ref[0:1, :] / float(_N) - mu * mu
        a = g_ref[...] * lax.rsqrt(var + _EPS)
        c = b_ref[...] - a * mu
        hn = jnp.maximum(pre_ref[...] * a + c, 0.0)
        if has_res:
            hn = hn + jnp.concatenate([res_ref[0], res_ref[1]], axis=1)
        stacked = jnp.stack([hn[:, :128], hn[:, 128:]], axis=0)
        h_ref[...] = stacked
        h16_ref[...] = stacked.astype(jnp.bfloat16)
        if has_q:
            q_ref[...] = jnp.dot(hn, wq_ref[...], preferred_element_type=jnp.float32)

    return body


def _layer_apply(pre, s1, s2, g, b, res=None, Wl3T=None):
    grid = _N // _R
    has_res = res is not None
    has_q = Wl3T is not None
    in_specs = [
        pl.BlockSpec((_R, _D), lambda i: (i, 0)),
        pl.BlockSpec((8, _D), lambda i: (0, 0)),
        pl.BlockSpec((8, _D), lambda i: (0, 0)),
        pl.BlockSpec((1, _D), lambda i: (0, 0)),
        pl.BlockSpec((1, _D), lambda i: (0, 0)),
    ]
    args = [pre, s1, s2, g, b]
    if has_res:
        in_specs.append(pl.BlockSpec((2, _R, 128), lambda i: (0, i, 0)))
        args.append(res)
    if has_q:
        in_specs.append(pl.BlockSpec((_D, _C), lambda i: (0, 0)))
        args.append(Wl3T)
    out_specs = [
        pl.BlockSpec((2, _R, 128), lambda i: (0, i, 0)),
        pl.BlockSpec((2, _R, 128), lambda i: (0, i, 0)),
    ]
    out_shape = [
        jax.ShapeDtypeStruct((2, _N, 128), jnp.float32),
        jax.ShapeDtypeStruct((2, _N, 128), jnp.bfloat16),
    ]
    if has_q:
        out_specs.append(pl.BlockSpec((_R, _C), lambda i: (i, 0)))
        out_shape.append(jax.ShapeDtypeStruct((_N, _C), jnp.float32))
    return pl.pallas_call(
        _make_apply_body(has_res, has_q),
        grid=(grid,),
        in_specs=in_specs,
        out_specs=out_specs,
        out_shape=out_shape,
    )(*args)


def _final_body(t_ref, c_ref, fw_ref, out_ref):
    cnt = c_ref[0, :, 0:1] + c_ref[1, :, 0:1]
    recip = 1.0 / jnp.maximum(cnt, 1.0)
    tsum = t_ref[0] + t_ref[1]
    out_ref[...] = tsum * recip + fw_ref[...]


def _layer_final(T2, cnt2, fW):
    grid = _N // _R
    return pl.pallas_call(
        _final_body,
        grid=(grid,),
        in_specs=[
            pl.BlockSpec((2, _R, 16), lambda i: (0, i, 0)),
            pl.BlockSpec((2, _R, 16), lambda i: (0, i, 0)),
            pl.BlockSpec((_R, _C), lambda i: (i, 0)),
        ],
        out_specs=pl.BlockSpec((_R, _C), lambda i: (i, 0)),
        out_shape=jax.ShapeDtypeStruct((_N, _C), jnp.float32),
    )(T2, cnt2, fW)


def kernel(x, edge_index, Wl0, bl0, Wr0, Wl1, bl1, Wr1, Wl2, bl2, Wr2, Wl3, bl3, Wr3, g0, b0, g1, b1, g2, b2):
    f32 = jnp.float32
    src = edge_index[0]
    dst = edge_index[1]
    pad16 = 16 * _EPT16 - _E
    pad32 = 32 * _EPT32 - _E
    srcp = jnp.concatenate([src, jnp.zeros((pad16,), jnp.int32)])
    dstp = jnp.concatenate([dst, jnp.full((pad16,), _N, jnp.int32)])
    s16 = srcp.reshape(16, _NCH16, _KA)
    src3 = jnp.concatenate([s16, s16 + _N], axis=0).astype(jnp.int16)
    dst3 = dstp.reshape(16, _NCH16, _KA).astype(jnp.int16)
    src32 = jnp.concatenate([src, jnp.zeros((pad32,), jnp.int32)]).reshape(32, _NCH32, _KB)
    dst32 = jnp.concatenate([dst, jnp.full((pad32,), _N, jnp.int32)]).reshape(32, _NCH32, _KB)
    zrows = jnp.zeros((_RPT, 128), f32)
    z16 = jnp.zeros((_RPT, 16), f32)
    ones16 = jnp.ones((_KB, 16), f32)

    cnt2 = _make_counts()(dst32, ones16, z16).reshape(2, _NP, 16)

    perm = jnp.asarray(_PERM)
    WlTs = [Wl0.T[perm], Wl1.T[perm], Wl2.T[perm]]
    WrTs = [Wr0.T, Wr1.T, Wr2.T]
    bls = [bl0.reshape(1, _D), bl1.reshape(1, _D), bl2.reshape(1, _D)]
    gs = [g0.reshape(1, _D), g1.reshape(1, _D), g2.reshape(1, _D)]
    bs = [b0.reshape(1, _D), b1.reshape(1, _D), b2.reshape(1, _D)]

    h = jnp.stack([x[:, :128], x[:, 128:]], axis=0)
    h16 = h.astype(jnp.bfloat16)
    q = None
    for l in range(3):
        S2 = _make_agg128()(h16.reshape(2 * _N, 128), src3, dst3, zrows).reshape(2, _NP, 128)
        preW = _layer_prew(h, WrTs[l], bls[l])
        pre, s1, s2 = _layer_pre(S2, cnt2, preW, WlTs[l])
        if l < 2:
            h, h16 = _layer_apply(pre, s1, s2, gs[l], bs[l], res=(h if l > 0 else None))
        else:
            h, h16, q = _layer_apply(pre, s1, s2, gs[l], bs[l], res=h, Wl3T=Wl3.T)

    T2 = _make_agg16()(q, src32, dst32, z16).reshape(2, _NP, 16)
    fW = _layer_prew(h, Wr3.T, bl3.reshape(1, _C))
    return _layer_final(T2, cnt2, fW)
